# Initial kernel scaffold; baseline (speedup 1.0000x reference)
import jax
import jax.numpy as jnp
from jax.experimental import pallas as pl


def kernel(token_ids, feature_ids, type_ids, pos_ids, seq_name_ids, token_table, feature_table, type_table, seq_table, pos_table):
    raise NotImplementedError("write your pallas kernel here")


# speedup vs baseline: 5.5094x; 5.5094x over previous
import functools

import jax
import jax.numpy as jnp
from jax import lax
from jax.experimental import pallas as pl
from jax.experimental.pallas import tpu as pltpu
from jax.experimental.pallas import tpu_sc as plsc

D = 64
T = 4
E = 128
LANES = 16


def _sc_kernel_body(num_chunks,
                    tok_ids, feat_ids, typ_ids, pos_ids, seq_ids,
                    token_table, feature_table, type_table, seq_table,
                    pos_table, out_hbm,
                    i_tok0, i_tok1, i_tok2, i_tok3,
                    i_feat0, i_feat1, i_feat2, i_feat3,
                    i_typ0, i_typ1, i_typ2, i_typ3,
                    idx_pos, idx_seq,
                    wbuf,
                    rows_tok, rows_feat, rows_typ, rows_pos, rows_seq,
                    outb, sem):
    idx_tok = [i_tok0, i_tok1, i_tok2, i_tok3]
    idx_feat = [i_feat0, i_feat1, i_feat2, i_feat3]
    idx_typ = [i_typ0, i_typ1, i_typ2, i_typ3]
    nc = 2
    wid = lax.axis_index("s") * nc + lax.axis_index("c")
    base0 = wid * (num_chunks * E)

    def chunk_body(c, _):
        base = base0 + c * E

        for t in range(T):
            pltpu.sync_copy(tok_ids.at[t, pl.ds(base, E)], idx_tok[t])
            pltpu.sync_copy(feat_ids.at[t, pl.ds(base, E)], idx_feat[t])
            pltpu.sync_copy(typ_ids.at[t, pl.ds(base, E)], idx_typ[t])
        pltpu.sync_copy(pos_ids.at[pl.ds(base, E)], idx_pos)
        pltpu.sync_copy(seq_ids.at[pl.ds(base, E)], idx_seq)

        def mask_body(i, _):
            sl = pl.ds(i * LANES, LANES)
            ms = []
            cnt = jnp.zeros((LANES,), jnp.float32)
            for t in range(T):
                m = jnp.where(idx_tok[t][sl] != 0, 1.0, 0.0).astype(jnp.float32)
                ms.append(m)
                cnt = cnt + m
            rv = 1.0 / jnp.maximum(cnt, 1.0)
            for t in range(T):
                wbuf[pl.ds(t * E + i * LANES, LANES)] = ms[t] * rv
            return 0
        lax.fori_loop(0, E // LANES, mask_body, 0)

        copies = []
        for t in range(T):
            copies.append(pltpu.make_async_copy(
                token_table.at[idx_tok[t]], rows_tok.at[t], sem))
            copies.append(pltpu.make_async_copy(
                feature_table.at[idx_feat[t]], rows_feat.at[t], sem))
            copies.append(pltpu.make_async_copy(
                type_table.at[idx_typ[t]], rows_typ.at[t], sem))
        copies.append(pltpu.make_async_copy(pos_table.at[idx_pos], rows_pos, sem))
        copies.append(pltpu.make_async_copy(seq_table.at[idx_seq], rows_seq, sem))
        for cp in copies:
            cp.start()
        for cp in copies:
            cp.wait()

        def event_body(e, _):
            w = [wbuf[pl.ds(t * E + e, LANES)][0] for t in range(T)]
            for dc in range(D // LANES):
                sl = pl.ds(dc * LANES, LANES)
                acc = rows_pos[e, sl] + rows_seq[e, sl]
                for t in range(T):
                    acc = acc + (rows_tok[t, e, sl] + rows_feat[t, e, sl]
                                 + rows_typ[t, e, sl]) * w[t]
                outb[e, sl] = acc
            return 0
        lax.fori_loop(0, E, event_body, 0)

        pltpu.sync_copy(outb, out_hbm.at[pl.ds(base, E), :])
        return 0

    lax.fori_loop(0, num_chunks, chunk_body, 0)


@functools.partial(jax.jit, static_argnums=())
def kernel(token_ids, feature_ids, type_ids, pos_ids, seq_name_ids,
           token_table, feature_table, type_table, seq_table, pos_table):
    B, S, L, Tt = token_ids.shape
    assert Tt == T and token_table.shape[1] == D
    N = B * S * L
    W = 32
    assert N % (W * E) == 0
    num_chunks = N // (W * E)

    tok_t = token_ids.reshape(N, T).T.astype(jnp.int32)
    feat_t = feature_ids.reshape(N, T).T.astype(jnp.int32)
    typ_t = type_ids.reshape(N, T).T.astype(jnp.int32)
    pos_f = pos_ids.reshape(N).astype(jnp.int32)
    seq_f = seq_name_ids.reshape(N).astype(jnp.int32)

    mesh = plsc.VectorSubcoreMesh(core_axis_name="c", subcore_axis_name="s",
                                  num_cores=2, num_subcores=16)
    kfn = pl.kernel(
        functools.partial(_sc_kernel_body, num_chunks),
        out_type=jax.ShapeDtypeStruct((N, D), jnp.float32),
        mesh=mesh,
        compiler_params=pltpu.CompilerParams(use_tc_tiling_on_sc=False),
        scratch_types=[
            *([pltpu.VMEM((E,), jnp.int32)] * 12),
            pltpu.VMEM((E,), jnp.int32),
            pltpu.VMEM((E,), jnp.int32),
            pltpu.VMEM((T * E + LANES,), jnp.float32),
            pltpu.VMEM((T, E, D), jnp.float32),
            pltpu.VMEM((T, E, D), jnp.float32),
            pltpu.VMEM((T, E, D), jnp.float32),
            pltpu.VMEM((E, D), jnp.float32),
            pltpu.VMEM((E, D), jnp.float32),
            pltpu.VMEM((E, D), jnp.float32),
            pltpu.SemaphoreType.DMA,
        ],
    )
    out = kfn(tok_t, feat_t, typ_t, pos_f, seq_f,
              token_table, feature_table, type_table, seq_table, pos_table)
    return out.reshape(B, S, L, D)
